# baseline (speedup 1.0000x reference)
import functools

import jax
import jax.numpy as jnp
from jax import lax
from jax.experimental import pallas as pl
from jax.experimental.pallas import tpu as pltpu
from jax.experimental.pallas import tpu_sc as plsc

VOCAB = 100000
HID = 128
B = 1024
S = 200
EPS = 1e-12

NW = 32
NTOK = B * S
TOK_PER_W = NTOK // NW
CHUNK = 128
NCHUNK = TOK_PER_W // CHUNK
NG = HID // 16
INV_HID = 1.0 / HID


def _group16(buf, ptab_v, pvec, gs, bs, red, g):
    lanes = lax.iota(jnp.int32, 16)
    for k in range(16):
        r = 16 * g + k
        p = pvec[k]
        acc1 = None
        acc2 = None
        for w in range(NG):
            x = buf[r, pl.ds(16 * w, 16)] + ptab_v[p, pl.ds(16 * w, 16)]
            buf[r, pl.ds(16 * w, 16)] = x
            acc1 = x if acc1 is None else acc1 + x
            acc2 = x * x if acc2 is None else acc2 + x * x
        kf = jnp.full((16,), k, dtype=jnp.int32)
        plsc.store_scatter(red, [lanes, kf], acc1)
        plsc.store_scatter(red, [lanes, 16 + kf], acc2)
    s1 = red[0, pl.ds(0, 16)]
    s2 = red[0, pl.ds(16, 16)]
    for j in range(1, 16):
        s1 = s1 + red[j, pl.ds(0, 16)]
        s2 = s2 + red[j, pl.ds(16, 16)]
    mu = s1 * INV_HID
    var = s2 * INV_HID - mu * mu + EPS
    iv = lax.bitcast_convert_type(var, jnp.int32)
    y = lax.bitcast_convert_type(0x5F3759DF - (iv >> 1), jnp.float32)
    for _ in range(3):
        y = y * (1.5 - 0.5 * var * y * y)
    c = mu * y
    for k in range(16):
        r = 16 * g + k
        a_k = y[k]
        c_k = c[k]
        for w in range(NG):
            t = buf[r, pl.ds(16 * w, 16)] * a_k - c_k
            buf[r, pl.ds(16 * w, 16)] = t * gs[w] + bs[w]


def _fused_kernel(ids_hbm, prow_hbm, table_hbm, ptab_hbm, gb_hbm, out_hbm,
                  idx_v, prow_v, buf0, buf1, ptab_v, gb_v, red, sem0, sem1):
    wid = lax.axis_index("s") * 2 + lax.axis_index("c")
    base = wid * TOK_PER_W
    pltpu.sync_copy(ids_hbm.at[wid], idx_v)
    pltpu.sync_copy(prow_hbm.at[wid], prow_v)
    pltpu.sync_copy(ptab_hbm, ptab_v)
    pltpu.sync_copy(gb_hbm, gb_v)
    gs = [gb_v[0, pl.ds(16 * w, 16)] for w in range(NG)]
    bs = [gb_v[1, pl.ds(16 * w, 16)] for w in range(NG)]
    bufs = (buf0, buf1)
    sems = (sem0, sem1)

    def start(c, b):
        off = pl.multiple_of(c * CHUNK, CHUNK)
        pltpu.async_copy(
            table_hbm.at[idx_v.at[pl.ds(off, CHUNK)]], bufs[b], sems[b])

    def drain(b):
        pltpu.make_async_copy(
            table_hbm.at[pl.ds(0, CHUNK)], bufs[b], sems[b]).wait()

    start(0, 0)
    start(1, 1)

    def outer(i, carry):
        for b in range(2):
            c = i * 2 + b
            cbase = pl.multiple_of(c * CHUNK, CHUNK)
            drain(b)

            def tokgroup(g, inner_carry):
                pvec = prow_v[pl.ds(cbase + 16 * g, 16)]
                _group16(bufs[b], ptab_v, pvec, gs, bs, red, g)
                return inner_carry

            lax.fori_loop(0, CHUNK // 16, tokgroup, 0)
            pltpu.sync_copy(bufs[b],
                            out_hbm.at[pl.ds(base + cbase, CHUNK)])

            @pl.when(c + 2 < NCHUNK)
            def _():
                start(c + 2, b)
        return carry

    lax.fori_loop(0, NCHUNK // 2, outer, 0)


def _sc_fused(ids, prow, word_emb, ptab, gb):
    mesh = plsc.VectorSubcoreMesh(core_axis_name="c", subcore_axis_name="s")
    k = functools.partial(
        pl.kernel,
        mesh=mesh,
        compiler_params=pltpu.CompilerParams(needs_layout_passes=False),
        out_type=jax.ShapeDtypeStruct((NTOK, HID), jnp.float32),
        scratch_types=[
            pltpu.VMEM((TOK_PER_W,), jnp.int32),
            pltpu.VMEM((TOK_PER_W,), jnp.int32),
            pltpu.VMEM((CHUNK, HID), jnp.float32),
            pltpu.VMEM((CHUNK, HID), jnp.float32),
            pltpu.VMEM((2 * S, HID), jnp.float32),
            pltpu.VMEM((2, HID), jnp.float32),
            pltpu.VMEM((32, 32), jnp.float32),
            pltpu.SemaphoreType.DMA,
            pltpu.SemaphoreType.DMA,
        ],
    )(_fused_kernel)
    return k(ids, prow, word_emb, ptab, gb)


def kernel(input_ids, token_type_ids, word_emb, pos_emb, type_emb, gamma, beta):
    ids = input_ids.astype(jnp.int32).reshape(NW, TOK_PER_W)
    pos_ids = jnp.arange(S, dtype=jnp.int32)[None, :]
    prow = (token_type_ids.astype(jnp.int32) * S + pos_ids).reshape(NW, TOK_PER_W)
    ptab = (type_emb[:, None, :] + pos_emb[None, :S, :]).reshape(2 * S, HID)
    gb = jnp.stack([gamma, beta])
    rows = _sc_fused(ids, prow, word_emb, ptab, gb)
    return rows.reshape(B, S, HID)


# speedup vs baseline: 2.6087x; 2.6087x over previous
import functools

import jax
import jax.numpy as jnp
from jax import lax
from jax.experimental import pallas as pl
from jax.experimental.pallas import tpu as pltpu
from jax.experimental.pallas import tpu_sc as plsc

VOCAB = 100000
HID = 128
B = 1024
S = 200
EPS = 1e-12

NW = 32
SLABS = (128, 448, 320, 128)
CHUNK = 80
BB = 32


def _gather_body(tok_per_w, nchunk):
    def _gather_kernel(ids_hbm, table_hbm, out_hbm, idx_v, buf0, buf1,
                       sem0, sem1):
        wid = lax.axis_index("s") * 2 + lax.axis_index("c")
        base = wid * tok_per_w
        pltpu.sync_copy(ids_hbm.at[wid], idx_v)
        bufs = (buf0, buf1)
        sems = (sem0, sem1)

        def start(c):
            return pltpu.async_copy(
                table_hbm.at[idx_v.at[pl.ds(c * CHUNK, CHUNK)]],
                bufs[c % 2], sems[c % 2])

        handles = [None] * nchunk
        handles[0] = start(0)
        for c in range(nchunk):
            if c + 1 < nchunk:
                handles[c + 1] = start(c + 1)
            handles[c].wait()
            pltpu.sync_copy(bufs[c % 2],
                            out_hbm.at[pl.ds(base + c * CHUNK, CHUNK)])

    return _gather_kernel


def _sc_gather(ids, word_emb, sb):
    stok = sb * S
    tok_per_w = stok // NW
    nchunk = tok_per_w // CHUNK
    mesh = plsc.VectorSubcoreMesh(core_axis_name="c", subcore_axis_name="s")
    k = functools.partial(
        pl.kernel,
        mesh=mesh,
        out_type=jax.ShapeDtypeStruct((stok, HID), jnp.float32),
        scratch_types=[
            pltpu.VMEM((tok_per_w,), jnp.int32),
            pltpu.VMEM((CHUNK, HID), jnp.float32),
            pltpu.VMEM((CHUNK, HID), jnp.float32),
            pltpu.SemaphoreType.DMA,
            pltpu.SemaphoreType.DMA,
        ],
    )(_gather_body(tok_per_w, nchunk))
    return k(ids, word_emb)


def _ln_kernel(words_ref, tt_ref, pt0_ref, ptd_ref, gamma_ref, beta_ref,
               prev_ref, out_ref):
    del prev_ref
    words = words_ref[...]
    tt = tt_ref[:, 0, :].astype(jnp.float32)
    emb = (words + pt0_ref[...][None, :, :]
           + tt[:, :, None] * ptd_ref[0][None, None, :])
    mu = jnp.mean(emb, axis=-1, keepdims=True)
    xc = emb - mu
    var = jnp.mean(xc * xc, axis=-1, keepdims=True)
    y = xc * lax.rsqrt(var + EPS)
    out_ref[...] = y * gamma_ref[0][None, None, :] + beta_ref[0][None, None, :]


def _tc_add_ln(step_off, steps, words, token_type_ids, pt0, ptd, gamma, beta,
               prev):
    in_specs = [
        pl.BlockSpec((BB, S, HID), lambda i: (i, 0, 0)),
        pl.BlockSpec((BB, 1, S), lambda i: (i, 0, 0)),
        pl.BlockSpec((S, HID), lambda i: (0, 0)),
        pl.BlockSpec((1, HID), lambda i: (0, 0)),
        pl.BlockSpec((1, HID), lambda i: (0, 0)),
        pl.BlockSpec((1, HID), lambda i: (0, 0)),
    ]
    args = [words, token_type_ids, pt0, ptd, gamma, beta]
    aliases = {}
    body = _ln_kernel
    if prev is not None:
        in_specs.append(pl.BlockSpec(memory_space=pl.ANY))
        args.append(prev)
        aliases = {6: 0}
    else:
        body = functools.partial(
            lambda *refs: _ln_kernel(*refs[:6], None, refs[6]))
    return pl.pallas_call(
        body,
        grid=(steps,),
        in_specs=in_specs,
        out_specs=pl.BlockSpec(
            (BB, S, HID), lambda i, _o=step_off: (_o + i, 0, 0)),
        out_shape=jax.ShapeDtypeStruct((B, S, HID), jnp.float32),
        input_output_aliases=aliases,
    )(*args)


def kernel(input_ids, token_type_ids, word_emb, pos_emb, type_emb, gamma, beta):
    ids_flat = input_ids.astype(jnp.int32).reshape(-1)
    tt = token_type_ids.astype(jnp.int32).reshape(B, 1, S)
    pt0 = pos_emb[:S] + type_emb[0][None, :]
    ptd = (type_emb[1] - type_emb[0]).reshape(1, HID)
    g2 = gamma.reshape(1, HID)
    b2 = beta.reshape(1, HID)
    slab_words = []
    row = 0
    for sb in SLABS:
        stok = sb * S
        ids_s = lax.dynamic_slice_in_dim(ids_flat, row * S, stok).reshape(
            NW, stok // NW)
        slab_words.append(_sc_gather(ids_s, word_emb, sb).reshape(sb, S, HID))
        row += sb
    out = None
    row = 0
    for sb, words in zip(SLABS, slab_words):
        out = _tc_add_ln(row // BB, sb // BB, words,
                         tt[row:row + sb], pt0, ptd, g2, b2, out)
        row += sb
    return out.reshape(B, S, HID)
